# baseline (speedup 1.0000x reference)
import functools

import jax
import jax.numpy as jnp
from jax import lax
from jax.experimental import pallas as pl
from jax.experimental.pallas import tpu as pltpu
from jax.experimental.pallas import tpu_sc as plsc

N = 10000
E = 320000
H = 128
G = 64
BN = 1000
BE = 2560
C = 32
CHUNKS = E // C
KMAX = CHUNKS // 16

_f32 = jnp.float32


def _dot(a, b):
    return jnp.dot(a, b, preferred_element_type=_f32)


def _tc_params():
    return pltpu.CompilerParams(dimension_semantics=("arbitrary",))


def _head_body(h_ref, p_ref, whe, wpe, wgs, wgr, whsh, whsp, wps, whrh, whrp,
               bias, h0_ref, p0_ref, gr_ref, t0_ref, t1_ref, hb_ref):
    hb = h_ref[...]
    pb = p_ref[...]
    h0 = _dot(hb, whe[...]) + bias[0:1, :]
    p0 = _dot(pb, wpe[...]) + bias[1:2, :]
    h0_ref[...] = h0
    p0_ref[...] = p0
    gs = _dot(h0, wgs[...])
    gr_ref[...] = _dot(h0, wgr[...])
    t0_ref[:, :H] = gs
    t0_ref[:, H:] = _dot(h0, whsh[...]) + _dot(p0, whsp[...]) + bias[2:3, :]
    t1_ref[:, :H] = gs
    t1_ref[:, H:] = _dot(p0, wps[...]) + bias[3:4, :]
    hb_ref[...] = _dot(h0, whrh[...]) + _dot(p0, whrp[...]) + bias[4:5, :]


def _run_head(h, p, whe, wpe, wgs, wgr, whsh, whsp, wps, whrh, whrp, bias):
    nblk = pl.BlockSpec((BN, H), lambda i: (i, 0))
    tblk = pl.BlockSpec((BN, 2 * H), lambda i: (i, 0))
    pblk = pl.BlockSpec((BN, 16), lambda i: (i, 0))
    w128 = pl.BlockSpec((H, H), lambda i: (0, 0))
    w16 = pl.BlockSpec((16, H), lambda i: (0, 0))
    bblk = pl.BlockSpec((8, H), lambda i: (0, 0))
    out = jax.ShapeDtypeStruct((N, H), _f32)
    out2 = jax.ShapeDtypeStruct((N, 2 * H), _f32)
    return pl.pallas_call(
        _head_body,
        grid=(N // BN,),
        in_specs=[nblk, pblk, w128, w16, w128, w128, w128, w128, w128, w128,
                  w128, bblk],
        out_specs=[nblk, nblk, nblk, tblk, tblk, nblk],
        out_shape=[out, out, out, out2, out2, out],
        compiler_params=_tc_params(),
    )(h, p, whe, wpe, wgs, wgr, whsh, whsp, wps, whrh, whrp, bias)


def _edge_emb_body(e_ref, wee, wge0, bias, ee0_ref):
    combo = _dot(wee[...], wge0[...])
    cb = _dot(bias[0:1, :], wge0[...]) + bias[1:2, :]
    ee0_ref[...] = _dot(e_ref[...], combo) + cb


def _run_edge_emb(e, wee, wge0, bias):
    eblk16 = pl.BlockSpec((BE, 16), lambda i: (i, 0))
    eblk = pl.BlockSpec((BE, H), lambda i: (i, 0))
    w16 = pl.BlockSpec((16, H), lambda i: (0, 0))
    w128 = pl.BlockSpec((H, H), lambda i: (0, 0))
    bblk = pl.BlockSpec((8, H), lambda i: (0, 0))
    return pl.pallas_call(
        _edge_emb_body,
        grid=(E // BE,),
        in_specs=[eblk16, w16, w128, bblk],
        out_specs=eblk,
        out_shape=jax.ShapeDtypeStruct((E, H), _f32),
        compiler_params=_tc_params(),
    )(e, wee, wge0, bias)


def _mid_edge_body(eta_ref, e_ref, wee, wge1, bias, ee1_ref):
    combo = _dot(wee[...], wge1[...])
    cb = _dot(bias[0:1, :], wge1[...]) + bias[1:2, :]
    ee1_ref[...] = (_dot(jax.nn.relu(eta_ref[...]), wge1[...])
                    + _dot(e_ref[...], combo) + cb)


def _run_mid_edge(eta0, e, wee, wge1, bias):
    eblk = pl.BlockSpec((BE, H), lambda i: (i, 0))
    eblk16 = pl.BlockSpec((BE, 16), lambda i: (i, 0))
    w16 = pl.BlockSpec((16, H), lambda i: (0, 0))
    w128 = pl.BlockSpec((H, H), lambda i: (0, 0))
    bblk = pl.BlockSpec((8, H), lambda i: (0, 0))
    return pl.pallas_call(
        _mid_edge_body,
        grid=(E // BE,),
        in_specs=[eblk, eblk16, w16, w128, bblk],
        out_specs=eblk,
        out_shape=jax.ShapeDtypeStruct((E, H), _f32),
        compiler_params=_tc_params(),
    )(eta0, e, wee, wge1, bias)


def _mid_body(h0_ref, p0_ref, hb0_ref, aggh_ref, aggp_ref, wpr0, wgs, wgr,
              whsh, whsp, wps, whrh, whrp, bias,
              h1_ref, p1_ref, gr_ref, t0_ref, t1_ref, hb_ref):
    h_new = hb0_ref[...] + aggh_ref[...]
    p_new = _dot(h_new, wpr0[...]) + bias[0:1, :] + aggp_ref[...]
    h1 = h0_ref[...] + jax.nn.relu(h_new)
    p1 = p0_ref[...] + jnp.tanh(p_new)
    h1_ref[...] = h1
    p1_ref[...] = p1
    gs = _dot(h1, wgs[...])
    gr_ref[...] = _dot(h1, wgr[...])
    t0_ref[:, :H] = gs
    t0_ref[:, H:] = _dot(h1, whsh[...]) + _dot(p1, whsp[...]) + bias[1:2, :]
    t1_ref[:, :H] = gs
    t1_ref[:, H:] = _dot(p1, wps[...]) + bias[2:3, :]
    hb_ref[...] = _dot(h1, whrh[...]) + _dot(p1, whrp[...]) + bias[3:4, :]


def _run_mid(h0, p0, hb0, agg, wpr0, wgs, wgr, whsh, whsp, wps, whrh, whrp,
             bias):
    nblk = pl.BlockSpec((BN, H), lambda i: (i, 0))
    tblk = pl.BlockSpec((BN, 2 * H), lambda i: (i, 0))
    agg_h = pl.BlockSpec((BN, H), lambda i: (i, 0))
    agg_p = pl.BlockSpec((BN, H), lambda i: (i + N // BN, 0))
    w128 = pl.BlockSpec((H, H), lambda i: (0, 0))
    bblk = pl.BlockSpec((8, H), lambda i: (0, 0))
    out = jax.ShapeDtypeStruct((N, H), _f32)
    out2 = jax.ShapeDtypeStruct((N, 2 * H), _f32)
    return pl.pallas_call(
        _mid_body,
        grid=(N // BN,),
        in_specs=[nblk, nblk, nblk, agg_h, agg_p, w128, w128, w128, w128,
                  w128, w128, w128, w128, bblk],
        out_specs=[nblk, nblk, nblk, tblk, tblk, nblk],
        out_shape=[out, out, out, out2, out2, out],
        compiler_params=_tc_params(),
    )(h0, p0, hb0, agg, agg, wpr0, wgs, wgr, whsh, whsp, wps, whrh, whrp,
      bias)


def _tail_body(h1_ref, p1_ref, hb1_ref, aggh_ref, aggp_ref, batch_ref, wpr1,
               wr1h, wr1p, wr2, bias, out_ref, hsum, psum):
    i = pl.program_id(0)
    h_new = hb1_ref[...] + aggh_ref[...]
    p_new = _dot(h_new, wpr1[...]) + bias[0:1, :] + aggp_ref[...]
    h2 = h1_ref[...] + jax.nn.relu(h_new)
    p2 = p1_ref[...] + jnp.tanh(p_new)
    bvec = batch_ref[...]
    cols = lax.broadcasted_iota(jnp.int32, (BN, G), 1)
    onehot = jnp.where(bvec == cols, 1.0, 0.0).astype(_f32)
    dn = (((0,), (0,)), ((), ()))
    hpart = lax.dot_general(onehot, h2, dn, preferred_element_type=_f32)
    ppart = lax.dot_general(onehot, p2, dn, preferred_element_type=_f32)

    @pl.when(i == 0)
    def _():
        hsum[...] = hpart
        psum[...] = ppart

    @pl.when(i > 0)
    def _():
        hsum[...] += hpart
        psum[...] += ppart

    @pl.when(i == pl.num_programs(0) - 1)
    def _():
        z = jax.nn.relu(_dot(hsum[...], wr1h[...]) + _dot(psum[...], wr1p[...])
                        + bias[1:2, :])
        out_ref[...] = _dot(z, wr2[...]) + bias[2:3, :]


def _run_tail(h1, p1, hb1, agg, batch2, wpr1, wr1h, wr1p, wr2, bias):
    nblk = pl.BlockSpec((BN, H), lambda i: (i, 0))
    agg_h = pl.BlockSpec((BN, H), lambda i: (i, 0))
    agg_p = pl.BlockSpec((BN, H), lambda i: (i + N // BN, 0))
    batblk = pl.BlockSpec((BN, 1), lambda i: (i, 0))
    w128 = pl.BlockSpec((H, H), lambda i: (0, 0))
    bblk = pl.BlockSpec((8, H), lambda i: (0, 0))
    oblk = pl.BlockSpec((G, H), lambda i: (0, 0))
    return pl.pallas_call(
        _tail_body,
        grid=(N // BN,),
        in_specs=[nblk, nblk, nblk, agg_h, agg_p, batblk, w128, w128, w128,
                  w128, bblk],
        out_specs=oblk,
        out_shape=jax.ShapeDtypeStruct((G, H), _f32),
        scratch_shapes=[pltpu.VMEM((G, H), _f32), pltpu.VMEM((G, H), _f32)],
        compiler_params=_tc_params(),
    )(h1, p1, hb1, agg, agg, batch2, wpr1, wr1h, wr1p, wr2, bias)


def _make_sc_edge(write_eta):
    mesh = plsc.VectorSubcoreMesh(core_axis_name="c", subcore_axis_name="s")
    out_types = [jax.ShapeDtypeStruct((2 * N, H), _f32)]
    if write_eta:
        out_types.append(jax.ShapeDtypeStruct((E, H), _f32))
    scratch = [
        pltpu.VMEM((2, C), jnp.int32),
        pltpu.VMEM((2, C), jnp.int32),
        pltpu.VMEM((C, 2 * H), _f32),
        pltpu.VMEM((C, 2 * H), _f32),
        pltpu.VMEM((C, H), _f32),
        pltpu.VMEM((C, H), _f32),
        pltpu.VMEM((C, H), _f32),
        pltpu.VMEM((C, H), _f32),
        pltpu.VMEM((C, H), _f32),
    ]
    if write_eta:
        scratch.append(pltpu.VMEM((C, H), _f32))
    scratch += [
        pltpu.VMEM_SHARED((N, H), _f32),
        pltpu.SemaphoreType.DMA,
        pltpu.SemaphoreType.DMA,
    ]

    def body(send_hbm, rec_hbm, t0_hbm, t1_hbm, gr_hbm, ee_hbm,
             agg_hbm, eta_hbm, idx_s_v, idx_r_v, gm0, gm1, gr0, gr1,
             ee0, ee1, msg_v, eta_v, acc_sh, sem0, sem1):
        cid = lax.axis_index("c")
        sid = lax.axis_index("s")
        gm_b = (gm0, gm1)
        gr_b = (gr0, gr1)
        ee_b = (ee0, ee1)
        sem_b = (sem0, sem1)

        def zbody(i, carry):
            zero = jnp.zeros((16,), _f32)
            for j in range(8):
                msg_v[i, pl.ds(16 * j, 16)] = zero
            return carry

        lax.fori_loop(0, C, zbody, 0)
        nb = 1000
        base_rows = sid * nb

        @pl.when(sid < 10)
        def _():
            full = nb // C
            for kk in range(full):
                pltpu.sync_copy(msg_v,
                                acc_sh.at[pl.ds(base_rows + kk * C, C)])
            rem = nb - full * C
            if rem:
                pltpu.sync_copy(msg_v.at[pl.ds(0, rem)],
                                acc_sh.at[pl.ds(base_rows + full * C, rem)])

        plsc.subcore_barrier()

        def issue(kk, b):
            base = (kk * 16 + sid) * C
            pltpu.sync_copy(send_hbm.at[pl.ds(base, C)], idx_s_v.at[b])
            pltpu.sync_copy(rec_hbm.at[pl.ds(base, C)], idx_r_v.at[b])

            @pl.when(cid == 0)
            def _():
                pltpu.async_copy(t0_hbm.at[idx_s_v.at[b]], gm_b[b], sem_b[b])

            @pl.when(cid == 1)
            def _():
                pltpu.async_copy(t1_hbm.at[idx_s_v.at[b]], gm_b[b], sem_b[b])

            pltpu.async_copy(gr_hbm.at[idx_r_v.at[b]], gr_b[b], sem_b[b])
            pltpu.async_copy(ee_hbm.at[pl.ds(base, C)], ee_b[b], sem_b[b])

        def wait_inputs(b):
            pltpu.make_async_copy(t0_hbm.at[idx_s_v.at[b]], gm_b[b],
                                  sem_b[b]).wait()
            pltpu.make_async_copy(gr_hbm.at[idx_r_v.at[b]], gr_b[b],
                                  sem_b[b]).wait()
            pltpu.make_async_copy(ee_hbm.at[pl.ds(0, C)], ee_b[b],
                                  sem_b[b]).wait()

        def compute(kk, b):
            gm_v, gr_v, ee_v = gm_b[b], gr_b[b], ee_b[b]

            def edge_body(i, carry2):
                eh = []
                accv = None
                for j in range(8):
                    sl = pl.ds(16 * j, 16)
                    x = gm_v[i, sl] + gr_v[i, sl] + ee_v[i, sl]
                    v = 1.0 / (1.0 + jnp.exp(-x))
                    eh.append(v)
                    accv = v if accv is None else accv + v
                r = jnp.sum(accv)
                invv = 1.0 / jnp.broadcast_to(r, (16,))
                for j in range(8):
                    sl = pl.ds(16 * j, 16)
                    msg_v[i, sl] = (eh[j] * gm_v[i, pl.ds(H + 16 * j, 16)]) \
                        * invv
                    if write_eta:
                        eta_v[i, sl] = eh[j]
                return carry2

            lax.fori_loop(0, C, edge_body, 0)
            pltpu.sync_copy(msg_v, acc_sh.at[idx_r_v.at[b]], add=True)
            if write_eta:
                @pl.when(cid == 0)
                def _():
                    base = (kk * 16 + sid) * C
                    pltpu.sync_copy(eta_v, eta_hbm.at[pl.ds(base, C)])

        issue(0, 0)

        def outer(ko, carry):
            for b in range(2):
                kk = 2 * ko + b
                nxt = kk + 1

                @pl.when(nxt < KMAX)
                def _():
                    issue(nxt, 1 - b)

                wait_inputs(b)
                compute(kk, b)
            return carry

        lax.fori_loop(0, KMAX // 2, outer, 0)
        wait_inputs(0)
        compute(KMAX - 1, 0)

        plsc.subcore_barrier()

        @pl.when(sid < 10)
        def _():
            pltpu.sync_copy(acc_sh.at[pl.ds(base_rows, nb)],
                            agg_hbm.at[pl.ds(cid * N + base_rows, nb)])

    if write_eta:
        def fn(send_hbm, rec_hbm, t0_hbm, t1_hbm, gr_hbm,
               ee_hbm, agg_hbm, eta_hbm, i_s, i_r, gm0, gm1, gr0, gr1,
               ee0, ee1, msg_v, eta_v, acc_sh, sem0, sem1):
            body(send_hbm, rec_hbm, t0_hbm, t1_hbm, gr_hbm, ee_hbm,
                 agg_hbm, eta_hbm, i_s, i_r, gm0, gm1, gr0, gr1,
                 ee0, ee1, msg_v, eta_v, acc_sh, sem0, sem1)
    else:
        def fn(send_hbm, rec_hbm, t0_hbm, t1_hbm, gr_hbm,
               ee_hbm, agg_hbm, i_s, i_r, gm0, gm1, gr0, gr1,
               ee0, ee1, msg_v, acc_sh, sem0, sem1):
            body(send_hbm, rec_hbm, t0_hbm, t1_hbm, gr_hbm, ee_hbm,
                 agg_hbm, None, i_s, i_r, gm0, gm1, gr0, gr1,
                 ee0, ee1, msg_v, None, acc_sh, sem0, sem1)

    return pl.kernel(
        fn, out_type=out_types, mesh=mesh, scratch_types=scratch,
        compiler_params=pltpu.CompilerParams(needs_layout_passes=False))


def kernel(h, e, p, edge_index, batch, W_he, b_he, W_ee, b_ee, W_pe, b_pe,
           Wg, bg, Whs, bhs, Whr, bhr, Wps, bps, Wpr, bpr, Wr1, br1, Wr2,
           br2):
    send = edge_index[0].astype(jnp.int32)
    rec = edge_index[1].astype(jnp.int32)
    batch2 = batch.astype(jnp.int32).reshape(N, 1)
    z = jnp.zeros((H,), _f32)

    Wg_s = [Wg[l, :H] for l in range(2)]
    Wg_r = [Wg[l, H:2 * H] for l in range(2)]
    Wg_e = [Wg[l, 2 * H:] for l in range(2)]
    Whs_h = [Whs[l, :H] for l in range(2)]
    Whs_p = [Whs[l, H:] for l in range(2)]
    Whr_h = [Whr[l, :H] for l in range(2)]
    Whr_p = [Whr[l, H:] for l in range(2)]

    bias_head = jnp.stack([b_he, b_pe, bhs[0], bps[0], bhr[0], z, z, z])
    bias_ee0 = jnp.stack([b_ee, bg[0], z, z, z, z, z, z])
    bias_ee1 = jnp.stack([b_ee, bg[1], z, z, z, z, z, z])
    bias_mid = jnp.stack([bpr[0], bhs[1], bps[1], bhr[1], z, z, z, z])
    br2pad = jnp.pad(br2, (0, H - 1))
    bias_tail = jnp.stack([bpr[1], br1, br2pad, z, z, z, z, z])
    Wr2pad = jnp.pad(Wr2, ((0, 0), (0, H - 1)))

    h0, p0, gr0, t0_0, t1_0, hb0 = _run_head(
        h, p, W_he, W_pe, Wg_s[0], Wg_r[0], Whs_h[0], Whs_p[0], Wps[0],
        Whr_h[0], Whr_p[0], bias_head)
    ee0 = _run_edge_emb(e, W_ee, Wg_e[0], bias_ee0)

    agg0, eta0 = _make_sc_edge(True)(send, rec, t0_0, t1_0, gr0, ee0)

    h1, p1, gr1, t0_1, t1_1, hb1 = _run_mid(
        h0, p0, hb0, agg0, Wpr[0], Wg_s[1], Wg_r[1], Whs_h[1], Whs_p[1],
        Wps[1], Whr_h[1], Whr_p[1], bias_mid)
    ee1 = _run_mid_edge(eta0, e, W_ee, Wg_e[1], bias_ee1)

    agg1 = _make_sc_edge(False)(send, rec, t0_1, t1_1, gr1, ee1)
    if isinstance(agg1, (list, tuple)):
        agg1 = agg1[0]

    out = _run_tail(h1, p1, hb1, agg1, batch2, Wpr[1], Wr1[:H], Wr1[H:],
                    Wr2pad, bias_tail)
    return out[:, 0]


# speedup vs baseline: 2.0221x; 2.0221x over previous
import functools

import jax
import jax.numpy as jnp
from jax import lax
from jax.experimental import pallas as pl
from jax.experimental.pallas import tpu as pltpu
from jax.experimental.pallas import tpu_sc as plsc

N = 10000
E = 320000
H = 128
G = 64
BN = 1000
BE = 2560
C = 32
CHUNKS = E // C
KMAX = CHUNKS // 16

_f32 = jnp.float32


def _dot(a, b):
    return jnp.dot(a, b, preferred_element_type=_f32)


def _tc_params():
    return pltpu.CompilerParams(dimension_semantics=("arbitrary",))


def _head_body(h_ref, p_ref, whe, wpe, wgs, wgr, whsh, whsp, wps, whrh, whrp,
               bias, h0_ref, p0_ref, gs_ref, gr_ref, ms_ref, ps_ref, hb_ref):
    hb = h_ref[...]
    pb = p_ref[...]
    h0 = _dot(hb, whe[...]) + bias[0:1, :]
    p0 = _dot(pb, wpe[...]) + bias[1:2, :]
    h0_ref[...] = h0
    p0_ref[...] = p0
    gs_ref[...] = _dot(h0, wgs[...])
    gr_ref[...] = _dot(h0, wgr[...])
    ms_ref[...] = _dot(h0, whsh[...]) + _dot(p0, whsp[...]) + bias[2:3, :]
    ps_ref[...] = _dot(p0, wps[...]) + bias[3:4, :]
    hb_ref[...] = _dot(h0, whrh[...]) + _dot(p0, whrp[...]) + bias[4:5, :]


def _run_head(h, p, whe, wpe, wgs, wgr, whsh, whsp, wps, whrh, whrp, bias):
    nblk = pl.BlockSpec((BN, H), lambda i: (i, 0))
    pblk = pl.BlockSpec((BN, 16), lambda i: (i, 0))
    w128 = pl.BlockSpec((H, H), lambda i: (0, 0))
    w16 = pl.BlockSpec((16, H), lambda i: (0, 0))
    bblk = pl.BlockSpec((8, H), lambda i: (0, 0))
    out = jax.ShapeDtypeStruct((N, H), _f32)
    return pl.pallas_call(
        _head_body,
        grid=(N // BN,),
        in_specs=[nblk, pblk, w128, w16, w128, w128, w128, w128, w128, w128,
                  w128, bblk],
        out_specs=[nblk] * 7,
        out_shape=[out] * 7,
        compiler_params=_tc_params(),
    )(h, p, whe, wpe, wgs, wgr, whsh, whsp, wps, whrh, whrp, bias)


def _edge_emb_body(e_ref, wee, wge0, bias, ee0_ref):
    combo = _dot(wee[...], wge0[...])
    cb = _dot(bias[0:1, :], wge0[...]) + bias[1:2, :]
    ee0_ref[...] = _dot(e_ref[...], combo) + cb


def _run_edge_emb(e, wee, wge0, bias):
    eblk16 = pl.BlockSpec((BE, 16), lambda i: (i, 0))
    eblk = pl.BlockSpec((BE, H), lambda i: (i, 0))
    w16 = pl.BlockSpec((16, H), lambda i: (0, 0))
    w128 = pl.BlockSpec((H, H), lambda i: (0, 0))
    bblk = pl.BlockSpec((8, H), lambda i: (0, 0))
    return pl.pallas_call(
        _edge_emb_body,
        grid=(E // BE,),
        in_specs=[eblk16, w16, w128, bblk],
        out_specs=eblk,
        out_shape=jax.ShapeDtypeStruct((E, H), _f32),
        compiler_params=_tc_params(),
    )(e, wee, wge0, bias)


def _mid_edge_body(eta_ref, e_ref, wee, wge1, bias, ee1_ref):
    combo = _dot(wee[...], wge1[...])
    cb = _dot(bias[0:1, :], wge1[...]) + bias[1:2, :]
    ee1_ref[...] = (_dot(jax.nn.relu(eta_ref[...]), wge1[...])
                    + _dot(e_ref[...], combo) + cb)


def _run_mid_edge(eta0, e, wee, wge1, bias):
    eblk = pl.BlockSpec((BE, H), lambda i: (i, 0))
    eblk16 = pl.BlockSpec((BE, 16), lambda i: (i, 0))
    w16 = pl.BlockSpec((16, H), lambda i: (0, 0))
    w128 = pl.BlockSpec((H, H), lambda i: (0, 0))
    bblk = pl.BlockSpec((8, H), lambda i: (0, 0))
    return pl.pallas_call(
        _mid_edge_body,
        grid=(E // BE,),
        in_specs=[eblk, eblk16, w16, w128, bblk],
        out_specs=eblk,
        out_shape=jax.ShapeDtypeStruct((E, H), _f32),
        compiler_params=_tc_params(),
    )(eta0, e, wee, wge1, bias)


def _mid_body(h0_ref, p0_ref, hb0_ref, aggh_ref, aggp_ref, wpr0, wgs, wgr,
              whsh, whsp, wps, whrh, whrp, bias,
              h1_ref, p1_ref, gs_ref, gr_ref, ms_ref, ps_ref, hb_ref):
    h_new = hb0_ref[...] + aggh_ref[...]
    p_new = _dot(h_new, wpr0[...]) + bias[0:1, :] + aggp_ref[...]
    h1 = h0_ref[...] + jax.nn.relu(h_new)
    p1 = p0_ref[...] + jnp.tanh(p_new)
    h1_ref[...] = h1
    p1_ref[...] = p1
    gs_ref[...] = _dot(h1, wgs[...])
    gr_ref[...] = _dot(h1, wgr[...])
    ms_ref[...] = _dot(h1, whsh[...]) + _dot(p1, whsp[...]) + bias[1:2, :]
    ps_ref[...] = _dot(p1, wps[...]) + bias[2:3, :]
    hb_ref[...] = _dot(h1, whrh[...]) + _dot(p1, whrp[...]) + bias[3:4, :]


def _run_mid(h0, p0, hb0, agg, wpr0, wgs, wgr, whsh, whsp, wps, whrh, whrp,
             bias):
    nblk = pl.BlockSpec((BN, H), lambda i: (i, 0))
    agg_h = pl.BlockSpec((BN, H), lambda i: (i, 0))
    agg_p = pl.BlockSpec((BN, H), lambda i: (i + N // BN, 0))
    w128 = pl.BlockSpec((H, H), lambda i: (0, 0))
    bblk = pl.BlockSpec((8, H), lambda i: (0, 0))
    out = jax.ShapeDtypeStruct((N, H), _f32)
    return pl.pallas_call(
        _mid_body,
        grid=(N // BN,),
        in_specs=[nblk, nblk, nblk, agg_h, agg_p, w128, w128, w128, w128,
                  w128, w128, w128, w128, bblk],
        out_specs=[nblk] * 7,
        out_shape=[out] * 7,
        compiler_params=_tc_params(),
    )(h0, p0, hb0, agg, agg, wpr0, wgs, wgr, whsh, whsp, wps, whrh, whrp,
      bias)


def _tail_body(h1_ref, p1_ref, hb1_ref, aggh_ref, aggp_ref, batch_ref, wpr1,
               wr1h, wr1p, wr2, bias, out_ref, hsum, psum):
    i = pl.program_id(0)
    h_new = hb1_ref[...] + aggh_ref[...]
    p_new = _dot(h_new, wpr1[...]) + bias[0:1, :] + aggp_ref[...]
    h2 = h1_ref[...] + jax.nn.relu(h_new)
    p2 = p1_ref[...] + jnp.tanh(p_new)
    bvec = batch_ref[...]
    cols = lax.broadcasted_iota(jnp.int32, (BN, G), 1)
    onehot = jnp.where(bvec == cols, 1.0, 0.0).astype(_f32)
    dn = (((0,), (0,)), ((), ()))
    hpart = lax.dot_general(onehot, h2, dn, preferred_element_type=_f32)
    ppart = lax.dot_general(onehot, p2, dn, preferred_element_type=_f32)

    @pl.when(i == 0)
    def _():
        hsum[...] = hpart
        psum[...] = ppart

    @pl.when(i > 0)
    def _():
        hsum[...] += hpart
        psum[...] += ppart

    @pl.when(i == pl.num_programs(0) - 1)
    def _():
        z = jax.nn.relu(_dot(hsum[...], wr1h[...]) + _dot(psum[...], wr1p[...])
                        + bias[1:2, :])
        out_ref[...] = _dot(z, wr2[...]) + bias[2:3, :]


def _run_tail(h1, p1, hb1, agg, batch2, wpr1, wr1h, wr1p, wr2, bias):
    nblk = pl.BlockSpec((BN, H), lambda i: (i, 0))
    agg_h = pl.BlockSpec((BN, H), lambda i: (i, 0))
    agg_p = pl.BlockSpec((BN, H), lambda i: (i + N // BN, 0))
    batblk = pl.BlockSpec((BN, 1), lambda i: (i, 0))
    w128 = pl.BlockSpec((H, H), lambda i: (0, 0))
    bblk = pl.BlockSpec((8, H), lambda i: (0, 0))
    oblk = pl.BlockSpec((G, H), lambda i: (0, 0))
    return pl.pallas_call(
        _tail_body,
        grid=(N // BN,),
        in_specs=[nblk, nblk, nblk, agg_h, agg_p, batblk, w128, w128, w128,
                  w128, bblk],
        out_specs=oblk,
        out_shape=jax.ShapeDtypeStruct((G, H), _f32),
        scratch_shapes=[pltpu.VMEM((G, H), _f32), pltpu.VMEM((G, H), _f32)],
        compiler_params=_tc_params(),
    )(h1, p1, hb1, agg, agg, batch2, wpr1, wr1h, wr1p, wr2, bias)


def _make_sc_edge(write_eta):
    mesh = plsc.VectorSubcoreMesh(core_axis_name="c", subcore_axis_name="s")
    out_types = [jax.ShapeDtypeStruct((2 * N, H), _f32)]
    if write_eta:
        out_types.append(jax.ShapeDtypeStruct((E, H), _f32))
    scratch = [
        pltpu.VMEM((2, C), jnp.int32),
        pltpu.VMEM((2, C), jnp.int32),
        pltpu.VMEM((C, H), _f32),
        pltpu.VMEM((C, H), _f32),
        pltpu.VMEM((C, H), _f32),
        pltpu.VMEM((C, H), _f32),
        pltpu.VMEM((C, H), _f32),
        pltpu.VMEM((C, H), _f32),
        pltpu.VMEM((C, H), _f32),
        pltpu.VMEM((C, H), _f32),
        pltpu.VMEM((C, H), _f32),
    ]
    if write_eta:
        scratch.append(pltpu.VMEM((C, H), _f32))
    scratch += [
        pltpu.VMEM_SHARED((N, H), _f32),
        pltpu.SemaphoreType.DMA,
        pltpu.SemaphoreType.DMA,
    ]

    def body(send_hbm, rec_hbm, gs_hbm, ms_hbm, ps_hbm, gr_hbm, ee_hbm,
             agg_hbm, eta_hbm, idx_s_v, idx_r_v, gs0, gs1, mp0, mp1,
             gr0, gr1, ee0, ee1, msg_v, eta_v, acc_sh, sem0, sem1):
        cid = lax.axis_index("c")
        sid = lax.axis_index("s")
        gs_b = (gs0, gs1)
        mp_b = (mp0, mp1)
        gr_b = (gr0, gr1)
        ee_b = (ee0, ee1)
        sem_b = (sem0, sem1)

        def zbody(i, carry):
            zero = jnp.zeros((16,), _f32)
            for j in range(8):
                msg_v[i, pl.ds(16 * j, 16)] = zero
            return carry

        lax.fori_loop(0, C, zbody, 0)
        nb = 1000
        base_rows = sid * nb

        @pl.when(sid < 10)
        def _():
            full = nb // C
            for kk in range(full):
                pltpu.sync_copy(msg_v,
                                acc_sh.at[pl.ds(base_rows + kk * C, C)])
            rem = nb - full * C
            if rem:
                pltpu.sync_copy(msg_v.at[pl.ds(0, rem)],
                                acc_sh.at[pl.ds(base_rows + full * C, rem)])

        plsc.subcore_barrier()

        def issue(kk, b):
            base = (kk * 16 + sid) * C
            pltpu.sync_copy(send_hbm.at[pl.ds(base, C)], idx_s_v.at[b])
            pltpu.sync_copy(rec_hbm.at[pl.ds(base, C)], idx_r_v.at[b])

            pltpu.async_copy(gs_hbm.at[idx_s_v.at[b]], gs_b[b], sem_b[b])

            @pl.when(cid == 0)
            def _():
                pltpu.async_copy(ms_hbm.at[idx_s_v.at[b]], mp_b[b], sem_b[b])

            @pl.when(cid == 1)
            def _():
                pltpu.async_copy(ps_hbm.at[idx_s_v.at[b]], mp_b[b], sem_b[b])

            pltpu.async_copy(gr_hbm.at[idx_r_v.at[b]], gr_b[b], sem_b[b])
            pltpu.async_copy(ee_hbm.at[pl.ds(base, C)], ee_b[b], sem_b[b])

        def wait_inputs(b):
            pltpu.make_async_copy(gs_hbm.at[idx_s_v.at[b]], gs_b[b],
                                  sem_b[b]).wait()
            pltpu.make_async_copy(ms_hbm.at[idx_s_v.at[b]], mp_b[b],
                                  sem_b[b]).wait()
            pltpu.make_async_copy(gr_hbm.at[idx_r_v.at[b]], gr_b[b],
                                  sem_b[b]).wait()
            pltpu.make_async_copy(ee_hbm.at[pl.ds(0, C)], ee_b[b],
                                  sem_b[b]).wait()

        def compute(kk, b):
            gs_v, mp_v, gr_v, ee_v = gs_b[b], mp_b[b], gr_b[b], ee_b[b]

            @plsc.parallel_loop(0, C)
            def edge_body(i):
                eh = []
                accv = None
                for j in range(8):
                    sl = pl.ds(16 * j, 16)
                    x = gs_v[i, sl] + gr_v[i, sl] + ee_v[i, sl]
                    v = 1.0 / (1.0 + jnp.exp(-x))
                    eh.append(v)
                    accv = v if accv is None else accv + v
                r = jnp.sum(accv)
                invv = 1.0 / jnp.broadcast_to(r, (16,))
                for j in range(8):
                    sl = pl.ds(16 * j, 16)
                    msg_v[i, sl] = (eh[j] * mp_v[i, sl]) * invv
                    if write_eta:
                        eta_v[i, sl] = eh[j]

            pltpu.sync_copy(msg_v, acc_sh.at[idx_r_v.at[b]], add=True)
            if write_eta:
                @pl.when(cid == 0)
                def _():
                    base = (kk * 16 + sid) * C
                    pltpu.sync_copy(eta_v, eta_hbm.at[pl.ds(base, C)])

        issue(0, 0)

        def outer(ko, carry):
            for b in range(2):
                kk = 2 * ko + b
                nxt = kk + 1

                @pl.when(nxt < KMAX)
                def _():
                    issue(nxt, 1 - b)

                wait_inputs(b)
                compute(kk, b)
            return carry

        lax.fori_loop(0, KMAX // 2, outer, 0)
        wait_inputs(0)
        compute(KMAX - 1, 0)

        plsc.subcore_barrier()

        @pl.when(sid < 10)
        def _():
            pltpu.sync_copy(acc_sh.at[pl.ds(base_rows, nb)],
                            agg_hbm.at[pl.ds(cid * N + base_rows, nb)])

    if write_eta:
        def fn(send_hbm, rec_hbm, gs_hbm, ms_hbm, ps_hbm, gr_hbm,
               ee_hbm, agg_hbm, eta_hbm, i_s, i_r, gs0, gs1, mp0, mp1,
               gr0, gr1, ee0, ee1, msg_v, eta_v, acc_sh, sem0, sem1):
            body(send_hbm, rec_hbm, gs_hbm, ms_hbm, ps_hbm, gr_hbm, ee_hbm,
                 agg_hbm, eta_hbm, i_s, i_r, gs0, gs1, mp0, mp1,
                 gr0, gr1, ee0, ee1, msg_v, eta_v, acc_sh, sem0, sem1)
    else:
        def fn(send_hbm, rec_hbm, gs_hbm, ms_hbm, ps_hbm, gr_hbm,
               ee_hbm, agg_hbm, i_s, i_r, gs0, gs1, mp0, mp1,
               gr0, gr1, ee0, ee1, msg_v, acc_sh, sem0, sem1):
            body(send_hbm, rec_hbm, gs_hbm, ms_hbm, ps_hbm, gr_hbm, ee_hbm,
                 agg_hbm, None, i_s, i_r, gs0, gs1, mp0, mp1,
                 gr0, gr1, ee0, ee1, msg_v, None, acc_sh, sem0, sem1)

    return pl.kernel(
        fn, out_type=out_types, mesh=mesh, scratch_types=scratch,
        compiler_params=pltpu.CompilerParams(needs_layout_passes=False))


def kernel(h, e, p, edge_index, batch, W_he, b_he, W_ee, b_ee, W_pe, b_pe,
           Wg, bg, Whs, bhs, Whr, bhr, Wps, bps, Wpr, bpr, Wr1, br1, Wr2,
           br2):
    send = edge_index[0].astype(jnp.int32)
    rec = edge_index[1].astype(jnp.int32)
    batch2 = batch.astype(jnp.int32).reshape(N, 1)
    z = jnp.zeros((H,), _f32)

    Wg_s = [Wg[l, :H] for l in range(2)]
    Wg_r = [Wg[l, H:2 * H] for l in range(2)]
    Wg_e = [Wg[l, 2 * H:] for l in range(2)]
    Whs_h = [Whs[l, :H] for l in range(2)]
    Whs_p = [Whs[l, H:] for l in range(2)]
    Whr_h = [Whr[l, :H] for l in range(2)]
    Whr_p = [Whr[l, H:] for l in range(2)]

    bias_head = jnp.stack([b_he, b_pe, bhs[0], bps[0], bhr[0], z, z, z])
    bias_ee0 = jnp.stack([b_ee, bg[0], z, z, z, z, z, z])
    bias_ee1 = jnp.stack([b_ee, bg[1], z, z, z, z, z, z])
    bias_mid = jnp.stack([bpr[0], bhs[1], bps[1], bhr[1], z, z, z, z])
    br2pad = jnp.pad(br2, (0, H - 1))
    bias_tail = jnp.stack([bpr[1], br1, br2pad, z, z, z, z, z])
    Wr2pad = jnp.pad(Wr2, ((0, 0), (0, H - 1)))

    h0, p0, gs0, gr0, ms0, ps0, hb0 = _run_head(
        h, p, W_he, W_pe, Wg_s[0], Wg_r[0], Whs_h[0], Whs_p[0], Wps[0],
        Whr_h[0], Whr_p[0], bias_head)
    ee0 = _run_edge_emb(e, W_ee, Wg_e[0], bias_ee0)

    agg0, eta0 = _make_sc_edge(True)(send, rec, gs0, ms0, ps0, gr0, ee0)

    h1, p1, gs1, gr1, ms1, ps1, hb1 = _run_mid(
        h0, p0, hb0, agg0, Wpr[0], Wg_s[1], Wg_r[1], Whs_h[1], Whs_p[1],
        Wps[1], Whr_h[1], Whr_p[1], bias_mid)
    ee1 = _run_mid_edge(eta0, e, W_ee, Wg_e[1], bias_ee1)

    agg1 = _make_sc_edge(False)(send, rec, gs1, ms1, ps1, gr1, ee1)
    if isinstance(agg1, (list, tuple)):
        agg1 = agg1[0]

    out = _run_tail(h1, p1, hb1, agg1, batch2, Wpr[1], Wr1[:H], Wr1[H:],
                    Wr2pad, bias_tail)
    return out[:, 0]


# speedup vs baseline: 2.4037x; 1.1887x over previous
import functools

import jax
import jax.numpy as jnp
from jax import lax
from jax.experimental import pallas as pl
from jax.experimental.pallas import tpu as pltpu
from jax.experimental.pallas import tpu_sc as plsc

N = 10000
E = 320000
H = 128
G = 64
BN = 1000
BE = 2560
C = 32
CHUNKS = E // C
KMAX = CHUNKS // 16

_f32 = jnp.float32


def _dot(a, b):
    return jnp.dot(a, b, preferred_element_type=_f32)


def _tc_params():
    return pltpu.CompilerParams(dimension_semantics=("arbitrary",))


def _head_body(h_ref, p_ref, whe, wpe, wgs, wgr, whsh, whsp, wps, whrh, whrp,
               bias, h0_ref, p0_ref, gs_ref, gr_ref, ms_ref, ps_ref, hb_ref):
    hb = h_ref[...]
    pb = p_ref[...]
    h0 = _dot(hb, whe[...]) + bias[0:1, :]
    p0 = _dot(pb, wpe[...]) + bias[1:2, :]
    h0_ref[...] = h0
    p0_ref[...] = p0
    gs_ref[...] = _dot(h0, wgs[...])
    gr_ref[...] = _dot(h0, wgr[...])
    ms_ref[...] = _dot(h0, whsh[...]) + _dot(p0, whsp[...]) + bias[2:3, :]
    ps_ref[...] = _dot(p0, wps[...]) + bias[3:4, :]
    hb_ref[...] = _dot(h0, whrh[...]) + _dot(p0, whrp[...]) + bias[4:5, :]


def _run_head(h, p, whe, wpe, wgs, wgr, whsh, whsp, wps, whrh, whrp, bias):
    nblk = pl.BlockSpec((BN, H), lambda i: (i, 0))
    pblk = pl.BlockSpec((BN, 16), lambda i: (i, 0))
    w128 = pl.BlockSpec((H, H), lambda i: (0, 0))
    w16 = pl.BlockSpec((16, H), lambda i: (0, 0))
    bblk = pl.BlockSpec((8, H), lambda i: (0, 0))
    out = jax.ShapeDtypeStruct((N, H), _f32)
    return pl.pallas_call(
        _head_body,
        grid=(N // BN,),
        in_specs=[nblk, pblk, w128, w16, w128, w128, w128, w128, w128, w128,
                  w128, bblk],
        out_specs=[nblk] * 7,
        out_shape=[out] * 7,
        compiler_params=_tc_params(),
    )(h, p, whe, wpe, wgs, wgr, whsh, whsp, wps, whrh, whrp, bias)


def _edge_emb_body(e_ref, wee, wge0, bias, ee0_ref):
    combo = _dot(wee[...], wge0[...])
    cb = _dot(bias[0:1, :], wge0[...]) + bias[1:2, :]
    ee0_ref[...] = _dot(e_ref[...], combo) + cb


def _run_edge_emb(e, wee, wge0, bias):
    eblk16 = pl.BlockSpec((BE, 16), lambda i: (i, 0))
    eblk = pl.BlockSpec((BE, H), lambda i: (i, 0))
    w16 = pl.BlockSpec((16, H), lambda i: (0, 0))
    w128 = pl.BlockSpec((H, H), lambda i: (0, 0))
    bblk = pl.BlockSpec((8, H), lambda i: (0, 0))
    return pl.pallas_call(
        _edge_emb_body,
        grid=(E // BE,),
        in_specs=[eblk16, w16, w128, bblk],
        out_specs=eblk,
        out_shape=jax.ShapeDtypeStruct((E, H), _f32),
        compiler_params=_tc_params(),
    )(e, wee, wge0, bias)


def _mid_edge_body(eta_ref, e_ref, wee, wge1, bias, ee1_ref):
    combo = _dot(wee[...], wge1[...])
    cb = _dot(bias[0:1, :], wge1[...]) + bias[1:2, :]
    ee1_ref[...] = (_dot(jax.nn.relu(eta_ref[...]), wge1[...])
                    + _dot(e_ref[...], combo) + cb)


def _run_mid_edge(eta0, e, wee, wge1, bias):
    eblk = pl.BlockSpec((BE, H), lambda i: (i, 0))
    eblk16 = pl.BlockSpec((BE, 16), lambda i: (i, 0))
    w16 = pl.BlockSpec((16, H), lambda i: (0, 0))
    w128 = pl.BlockSpec((H, H), lambda i: (0, 0))
    bblk = pl.BlockSpec((8, H), lambda i: (0, 0))
    return pl.pallas_call(
        _mid_edge_body,
        grid=(E // BE,),
        in_specs=[eblk, eblk16, w16, w128, bblk],
        out_specs=eblk,
        out_shape=jax.ShapeDtypeStruct((E, H), _f32),
        compiler_params=_tc_params(),
    )(eta0, e, wee, wge1, bias)


def _mid_body(h0_ref, p0_ref, hb0_ref, aggh_ref, aggp_ref, wpr0, wgs, wgr,
              whsh, whsp, wps, whrh, whrp, bias,
              h1_ref, p1_ref, gs_ref, gr_ref, ms_ref, ps_ref, hb_ref):
    h_new = hb0_ref[...] + aggh_ref[...]
    p_new = _dot(h_new, wpr0[...]) + bias[0:1, :] + aggp_ref[...]
    h1 = h0_ref[...] + jax.nn.relu(h_new)
    p1 = p0_ref[...] + jnp.tanh(p_new)
    h1_ref[...] = h1
    p1_ref[...] = p1
    gs_ref[...] = _dot(h1, wgs[...])
    gr_ref[...] = _dot(h1, wgr[...])
    ms_ref[...] = _dot(h1, whsh[...]) + _dot(p1, whsp[...]) + bias[1:2, :]
    ps_ref[...] = _dot(p1, wps[...]) + bias[2:3, :]
    hb_ref[...] = _dot(h1, whrh[...]) + _dot(p1, whrp[...]) + bias[3:4, :]


def _run_mid(h0, p0, hb0, agg, wpr0, wgs, wgr, whsh, whsp, wps, whrh, whrp,
             bias):
    nblk = pl.BlockSpec((BN, H), lambda i: (i, 0))
    agg_h = pl.BlockSpec((BN, H), lambda i: (i, 0))
    agg_p = pl.BlockSpec((BN, H), lambda i: (i + N // BN, 0))
    w128 = pl.BlockSpec((H, H), lambda i: (0, 0))
    bblk = pl.BlockSpec((8, H), lambda i: (0, 0))
    out = jax.ShapeDtypeStruct((N, H), _f32)
    return pl.pallas_call(
        _mid_body,
        grid=(N // BN,),
        in_specs=[nblk, nblk, nblk, agg_h, agg_p, w128, w128, w128, w128,
                  w128, w128, w128, w128, bblk],
        out_specs=[nblk] * 7,
        out_shape=[out] * 7,
        compiler_params=_tc_params(),
    )(h0, p0, hb0, agg, agg, wpr0, wgs, wgr, whsh, whsp, wps, whrh, whrp,
      bias)


def _tail_body(h1_ref, p1_ref, hb1_ref, aggh_ref, aggp_ref, batch_ref, wpr1,
               wr1h, wr1p, wr2, bias, out_ref, hsum, psum):
    i = pl.program_id(0)
    h_new = hb1_ref[...] + aggh_ref[...]
    p_new = _dot(h_new, wpr1[...]) + bias[0:1, :] + aggp_ref[...]
    h2 = h1_ref[...] + jax.nn.relu(h_new)
    p2 = p1_ref[...] + jnp.tanh(p_new)
    bvec = batch_ref[...]
    cols = lax.broadcasted_iota(jnp.int32, (BN, G), 1)
    onehot = jnp.where(bvec == cols, 1.0, 0.0).astype(_f32)
    dn = (((0,), (0,)), ((), ()))
    hpart = lax.dot_general(onehot, h2, dn, preferred_element_type=_f32)
    ppart = lax.dot_general(onehot, p2, dn, preferred_element_type=_f32)

    @pl.when(i == 0)
    def _():
        hsum[...] = hpart
        psum[...] = ppart

    @pl.when(i > 0)
    def _():
        hsum[...] += hpart
        psum[...] += ppart

    @pl.when(i == pl.num_programs(0) - 1)
    def _():
        z = jax.nn.relu(_dot(hsum[...], wr1h[...]) + _dot(psum[...], wr1p[...])
                        + bias[1:2, :])
        out_ref[...] = _dot(z, wr2[...]) + bias[2:3, :]


def _run_tail(h1, p1, hb1, agg, batch2, wpr1, wr1h, wr1p, wr2, bias):
    nblk = pl.BlockSpec((BN, H), lambda i: (i, 0))
    agg_h = pl.BlockSpec((BN, H), lambda i: (i, 0))
    agg_p = pl.BlockSpec((BN, H), lambda i: (i + N // BN, 0))
    batblk = pl.BlockSpec((BN, 1), lambda i: (i, 0))
    w128 = pl.BlockSpec((H, H), lambda i: (0, 0))
    bblk = pl.BlockSpec((8, H), lambda i: (0, 0))
    oblk = pl.BlockSpec((G, H), lambda i: (0, 0))
    return pl.pallas_call(
        _tail_body,
        grid=(N // BN,),
        in_specs=[nblk, nblk, nblk, agg_h, agg_p, batblk, w128, w128, w128,
                  w128, bblk],
        out_specs=oblk,
        out_shape=jax.ShapeDtypeStruct((G, H), _f32),
        scratch_shapes=[pltpu.VMEM((G, H), _f32), pltpu.VMEM((G, H), _f32)],
        compiler_params=_tc_params(),
    )(h1, p1, hb1, agg, agg, batch2, wpr1, wr1h, wr1p, wr2, bias)


def _make_sc_edge(write_eta):
    mesh = plsc.VectorSubcoreMesh(core_axis_name="c", subcore_axis_name="s")
    out_types = [jax.ShapeDtypeStruct((2 * N, H), _f32)]
    if write_eta:
        out_types.append(jax.ShapeDtypeStruct((E, H), _f32))
    scratch = [
        pltpu.VMEM((2, C), jnp.int32),
        pltpu.VMEM((2, C), jnp.int32),
        pltpu.VMEM((C, H), _f32),
        pltpu.VMEM((C, H), _f32),
        pltpu.VMEM((C, H), _f32),
        pltpu.VMEM((C, H), _f32),
        pltpu.VMEM((C, H), _f32),
        pltpu.VMEM((C, H), _f32),
        pltpu.VMEM((C, H), _f32),
        pltpu.VMEM((C, H), _f32),
        pltpu.VMEM((C, H), _f32),
    ]
    if write_eta:
        scratch.append(pltpu.VMEM((C, H), _f32))
    scratch += [
        pltpu.VMEM_SHARED((N, H), _f32),
        pltpu.SemaphoreType.DMA,
        pltpu.SemaphoreType.DMA,
        pltpu.SemaphoreType.DMA,
        pltpu.SemaphoreType.DMA,
    ]

    def body(send_hbm, rec_hbm, gs_hbm, ms_hbm, ps_hbm, gr_hbm, ee_hbm,
             agg_hbm, eta_hbm, idx_s_v, idx_r_v, gs0, gs1, mp0, mp1,
             gr0, gr1, ee0, ee1, msg_v, eta_v, acc_sh, sem0, sem1,
             semi0, semi1):
        cid = lax.axis_index("c")
        sid = lax.axis_index("s")
        gs_b = (gs0, gs1)
        mp_b = (mp0, mp1)
        gr_b = (gr0, gr1)
        ee_b = (ee0, ee1)
        sem_b = (sem0, sem1)
        sem_i = (semi0, semi1)

        def zbody(i, carry):
            zero = jnp.zeros((16,), _f32)
            for j in range(8):
                msg_v[i, pl.ds(16 * j, 16)] = zero
            return carry

        lax.fori_loop(0, C, zbody, 0)
        nb = 1000
        base_rows = sid * nb

        @pl.when(sid < 10)
        def _():
            full = nb // C
            for kk in range(full):
                pltpu.sync_copy(msg_v,
                                acc_sh.at[pl.ds(base_rows + kk * C, C)])
            rem = nb - full * C
            if rem:
                pltpu.sync_copy(msg_v.at[pl.ds(0, rem)],
                                acc_sh.at[pl.ds(base_rows + full * C, rem)])

        plsc.subcore_barrier()

        def issue_idx(kk, b):
            base = (kk * 16 + sid) * C
            pltpu.async_copy(send_hbm.at[pl.ds(base, C)], idx_s_v.at[b],
                             sem_i[b])
            pltpu.async_copy(rec_hbm.at[pl.ds(base, C)], idx_r_v.at[b],
                             sem_i[b])

        def wait_idx(b):
            pltpu.make_async_copy(send_hbm.at[pl.ds(0, C)], idx_s_v.at[b],
                                  sem_i[b]).wait()
            pltpu.make_async_copy(rec_hbm.at[pl.ds(0, C)], idx_r_v.at[b],
                                  sem_i[b]).wait()

        def issue(kk, b):
            base = (kk * 16 + sid) * C
            pltpu.async_copy(gs_hbm.at[idx_s_v.at[b]], gs_b[b], sem_b[b])

            @pl.when(cid == 0)
            def _():
                pltpu.async_copy(ms_hbm.at[idx_s_v.at[b]], mp_b[b], sem_b[b])

            @pl.when(cid == 1)
            def _():
                pltpu.async_copy(ps_hbm.at[idx_s_v.at[b]], mp_b[b], sem_b[b])

            pltpu.async_copy(gr_hbm.at[idx_r_v.at[b]], gr_b[b], sem_b[b])
            pltpu.async_copy(ee_hbm.at[pl.ds(base, C)], ee_b[b], sem_b[b])

        def wait_inputs(b):
            pltpu.make_async_copy(gs_hbm.at[idx_s_v.at[b]], gs_b[b],
                                  sem_b[b]).wait()
            pltpu.make_async_copy(ms_hbm.at[idx_s_v.at[b]], mp_b[b],
                                  sem_b[b]).wait()
            pltpu.make_async_copy(gr_hbm.at[idx_r_v.at[b]], gr_b[b],
                                  sem_b[b]).wait()
            pltpu.make_async_copy(ee_hbm.at[pl.ds(0, C)], ee_b[b],
                                  sem_b[b]).wait()

        def compute(kk, b):
            gs_v, mp_v, gr_v, ee_v = gs_b[b], mp_b[b], gr_b[b], ee_b[b]

            @plsc.parallel_loop(0, C)
            def edge_body(i):
                eh = []
                accv = None
                for j in range(8):
                    sl = pl.ds(16 * j, 16)
                    x = gs_v[i, sl] + gr_v[i, sl] + ee_v[i, sl]
                    v = 1.0 / (1.0 + jnp.exp(-x))
                    eh.append(v)
                    accv = v if accv is None else accv + v
                r = jnp.sum(accv)
                invv = 1.0 / jnp.broadcast_to(r, (16,))
                for j in range(8):
                    sl = pl.ds(16 * j, 16)
                    msg_v[i, sl] = (eh[j] * mp_v[i, sl]) * invv
                    if write_eta:
                        eta_v[i, sl] = eh[j]

            pltpu.sync_copy(msg_v, acc_sh.at[idx_r_v.at[b]], add=True)
            if write_eta:
                @pl.when(cid == 0)
                def _():
                    base = (kk * 16 + sid) * C
                    pltpu.sync_copy(eta_v, eta_hbm.at[pl.ds(base, C)])

        issue_idx(0, 0)
        wait_idx(0)
        issue(0, 0)
        issue_idx(1, 1)

        def outer(ko, carry):
            for b in range(2):
                kk = 2 * ko + b
                nxt = kk + 1

                @pl.when(nxt < KMAX)
                def _():
                    wait_idx(1 - b)
                    issue(nxt, 1 - b)

                wait_inputs(b)
                compute(kk, b)

                @pl.when(kk + 2 < KMAX)
                def _():
                    issue_idx(kk + 2, b)

            return carry

        lax.fori_loop(0, KMAX // 2, outer, 0)
        wait_inputs(0)
        compute(KMAX - 1, 0)

        plsc.subcore_barrier()

        @pl.when(sid < 10)
        def _():
            pltpu.sync_copy(acc_sh.at[pl.ds(base_rows, nb)],
                            agg_hbm.at[pl.ds(cid * N + base_rows, nb)])

    if write_eta:
        def fn(send_hbm, rec_hbm, gs_hbm, ms_hbm, ps_hbm, gr_hbm,
               ee_hbm, agg_hbm, eta_hbm, i_s, i_r, gs0, gs1, mp0, mp1,
               gr0, gr1, ee0, ee1, msg_v, eta_v, acc_sh, sem0, sem1,
               semi0, semi1):
            body(send_hbm, rec_hbm, gs_hbm, ms_hbm, ps_hbm, gr_hbm, ee_hbm,
                 agg_hbm, eta_hbm, i_s, i_r, gs0, gs1, mp0, mp1,
                 gr0, gr1, ee0, ee1, msg_v, eta_v, acc_sh, sem0, sem1,
                 semi0, semi1)
    else:
        def fn(send_hbm, rec_hbm, gs_hbm, ms_hbm, ps_hbm, gr_hbm,
               ee_hbm, agg_hbm, i_s, i_r, gs0, gs1, mp0, mp1,
               gr0, gr1, ee0, ee1, msg_v, acc_sh, sem0, sem1,
               semi0, semi1):
            body(send_hbm, rec_hbm, gs_hbm, ms_hbm, ps_hbm, gr_hbm, ee_hbm,
                 agg_hbm, None, i_s, i_r, gs0, gs1, mp0, mp1,
                 gr0, gr1, ee0, ee1, msg_v, None, acc_sh, sem0, sem1,
                 semi0, semi1)

    return pl.kernel(
        fn, out_type=out_types, mesh=mesh, scratch_types=scratch,
        compiler_params=pltpu.CompilerParams(needs_layout_passes=False))


def kernel(h, e, p, edge_index, batch, W_he, b_he, W_ee, b_ee, W_pe, b_pe,
           Wg, bg, Whs, bhs, Whr, bhr, Wps, bps, Wpr, bpr, Wr1, br1, Wr2,
           br2):
    send = edge_index[0].astype(jnp.int32)
    rec = edge_index[1].astype(jnp.int32)
    batch2 = batch.astype(jnp.int32).reshape(N, 1)
    z = jnp.zeros((H,), _f32)

    Wg_s = [Wg[l, :H] for l in range(2)]
    Wg_r = [Wg[l, H:2 * H] for l in range(2)]
    Wg_e = [Wg[l, 2 * H:] for l in range(2)]
    Whs_h = [Whs[l, :H] for l in range(2)]
    Whs_p = [Whs[l, H:] for l in range(2)]
    Whr_h = [Whr[l, :H] for l in range(2)]
    Whr_p = [Whr[l, H:] for l in range(2)]

    bias_head = jnp.stack([b_he, b_pe, bhs[0], bps[0], bhr[0], z, z, z])
    bias_ee0 = jnp.stack([b_ee, bg[0], z, z, z, z, z, z])
    bias_ee1 = jnp.stack([b_ee, bg[1], z, z, z, z, z, z])
    bias_mid = jnp.stack([bpr[0], bhs[1], bps[1], bhr[1], z, z, z, z])
    br2pad = jnp.pad(br2, (0, H - 1))
    bias_tail = jnp.stack([bpr[1], br1, br2pad, z, z, z, z, z])
    Wr2pad = jnp.pad(Wr2, ((0, 0), (0, H - 1)))

    h0, p0, gs0, gr0, ms0, ps0, hb0 = _run_head(
        h, p, W_he, W_pe, Wg_s[0], Wg_r[0], Whs_h[0], Whs_p[0], Wps[0],
        Whr_h[0], Whr_p[0], bias_head)
    ee0 = _run_edge_emb(e, W_ee, Wg_e[0], bias_ee0)

    agg0, eta0 = _make_sc_edge(True)(send, rec, gs0, ms0, ps0, gr0, ee0)

    h1, p1, gs1, gr1, ms1, ps1, hb1 = _run_mid(
        h0, p0, hb0, agg0, Wpr[0], Wg_s[1], Wg_r[1], Whs_h[1], Whs_p[1],
        Wps[1], Whr_h[1], Whr_p[1], bias_mid)
    ee1 = _run_mid_edge(eta0, e, W_ee, Wg_e[1], bias_ee1)

    agg1 = _make_sc_edge(False)(send, rec, gs1, ms1, ps1, gr1, ee1)
    if isinstance(agg1, (list, tuple)):
        agg1 = agg1[0]

    out = _run_tail(h1, p1, hb1, agg1, batch2, Wpr[1], Wr1[:H], Wr1[H:],
                    Wr2pad, bias_tail)
    return out[:, 0]
